# baseline (speedup 1.0000x reference)
import jax
import jax.numpy as jnp
from jax.experimental import pallas as pl

_B = 16
_BLK = 4
_THR = 0.5


def _body(t0c8, t1c8, t0c16, t1c16, t0c32,
          b0_8, c0_8, b1_8, c1_8,
          b0_16, c0_16, b1_16, c1_16,
          b0_32, c0_32,
          ob0_8, oc0_8, ob1_8, oc1_8,
          ob0_16, oc0_16, ob1_16, oc1_16,
          ob0_32, oc0_32):
    ones80 = jnp.ones((1, 80), jnp.float32)
    ones8 = jnp.ones((1, 8), jnp.float32)

    for i in range(_BLK):
        def mask_of(c0, c1, h, w):
            ind0 = (c0[i] > _THR).astype(jnp.float32)
            s = jax.lax.dot_general(ones80, ind0, (((1,), (1,)), ((), ())),
                                    preferred_element_type=jnp.float32)
            if c1 is not None:
                ind1 = (c1[i] > _THR).astype(jnp.float32)
                s = s + jax.lax.dot_general(ones8, ind1,
                                            (((1,), (0,)), ((), ())),
                                            preferred_element_type=jnp.float32)
            return (s > 0.0).astype(jnp.float32).reshape(1, h, w)

        m8 = mask_of(t0c8, t1c8, 64, 64)
        m16 = mask_of(t0c16, t1c16, 32, 32)
        m32 = mask_of(t0c32, None, 16, 16)

        for src, dst, m in ((b0_8, ob0_8, m8), (c0_8, oc0_8, m8),
                            (b1_8, ob1_8, m8), (c1_8, oc1_8, m8),
                            (b0_16, ob0_16, m16), (c0_16, oc0_16, m16),
                            (b1_16, ob1_16, m16), (c1_16, oc1_16, m16),
                            (b0_32, ob0_32, m32), (c0_32, oc0_32, m32)):
            dst[i] = src[i] * m


def kernel(t0_cls_s8, t0_cls_s16, t0_cls_s32,
           t0_box_s8, t0_box_s16, t0_box_s32,
           t0_ctr_s8, t0_ctr_s16, t0_ctr_s32,
           t1_cls_s8, t1_cls_s16,
           t1_box_s8, t1_box_s16,
           t1_ctr_s8, t1_ctr_s16):
    def cm(x):
        n, c, h, w = x.shape
        return x.transpose(0, 2, 3, 1).reshape(n, h * w, c)

    def flat(x):
        n, c, h, w = x.shape
        return x.reshape(n, c, h * w)

    ins = [cm(t0_cls_s8), flat(t1_cls_s8),
           cm(t0_cls_s16), flat(t1_cls_s16),
           cm(t0_cls_s32),
           t0_box_s8, t0_ctr_s8, t1_box_s8, t1_ctr_s8,
           t0_box_s16, t0_ctr_s16, t1_box_s16, t1_ctr_s16,
           t0_box_s32, t0_ctr_s32]
    in_specs = [pl.BlockSpec((_BLK,) + x.shape[1:],
                             (lambda n: (n, 0, 0)) if x.ndim == 3
                             else (lambda n: (n, 0, 0, 0))) for x in ins]
    out_shapes = [jax.ShapeDtypeStruct(x.shape, jnp.float32) for x in ins[5:]]
    out_specs = [pl.BlockSpec((_BLK,) + x.shape[1:], lambda n: (n, 0, 0, 0))
                 for x in ins[5:]]

    prods = pl.pallas_call(
        _body,
        grid=(_B // _BLK,),
        in_specs=in_specs,
        out_specs=out_specs,
        out_shape=out_shapes,
    )(*ins)
    prods = jax.lax.optimization_barrier(tuple(prods))

    result = []
    for g in (prods[0:4], prods[4:8], prods[8:10]):
        for n in range(_B):
            for o in g:
                result.append(o[n])
    return tuple(result)


# speedup vs baseline: 1.0094x; 1.0094x over previous
import jax
import jax.numpy as jnp
from jax.experimental import pallas as pl

_B = 16
_BLK = 2
_THR = 0.5


def _body(t0c8, t1c8, t0c16, t1c16, t0c32,
          b0_8, c0_8, b1_8, c1_8,
          b0_16, c0_16, b1_16, c1_16,
          b0_32, c0_32,
          ob0_8, oc0_8, ob1_8, oc1_8,
          ob0_16, oc0_16, ob1_16, oc1_16,
          ob0_32, oc0_32):
    ones80 = jnp.ones((1, 80), jnp.float32)
    ones8 = jnp.ones((1, 8), jnp.float32)

    for i in range(_BLK):
        def mask_of(c0, c1, h, w):
            ind0 = (c0[i] > _THR).astype(jnp.float32)
            s = jax.lax.dot_general(ones80, ind0, (((1,), (1,)), ((), ())),
                                    preferred_element_type=jnp.float32)
            if c1 is not None:
                ind1 = (c1[i] > _THR).astype(jnp.float32)
                s = s + jax.lax.dot_general(ones8, ind1,
                                            (((1,), (0,)), ((), ())),
                                            preferred_element_type=jnp.float32)
            return (s > 0.0).astype(jnp.float32).reshape(1, h, w)

        m8 = mask_of(t0c8, t1c8, 64, 64)
        m16 = mask_of(t0c16, t1c16, 32, 32)
        m32 = mask_of(t0c32, None, 16, 16)

        for src, dst, m in ((b0_8, ob0_8, m8), (c0_8, oc0_8, m8),
                            (b1_8, ob1_8, m8), (c1_8, oc1_8, m8),
                            (b0_16, ob0_16, m16), (c0_16, oc0_16, m16),
                            (b1_16, ob1_16, m16), (c1_16, oc1_16, m16),
                            (b0_32, ob0_32, m32), (c0_32, oc0_32, m32)):
            dst[i] = src[i] * m


def kernel(t0_cls_s8, t0_cls_s16, t0_cls_s32,
           t0_box_s8, t0_box_s16, t0_box_s32,
           t0_ctr_s8, t0_ctr_s16, t0_ctr_s32,
           t1_cls_s8, t1_cls_s16,
           t1_box_s8, t1_box_s16,
           t1_ctr_s8, t1_ctr_s16):
    def cm(x):
        n, c, h, w = x.shape
        return x.transpose(0, 2, 3, 1).reshape(n, h * w, c)

    def flat(x):
        n, c, h, w = x.shape
        return x.reshape(n, c, h * w)

    ins = [cm(t0_cls_s8), flat(t1_cls_s8),
           cm(t0_cls_s16), flat(t1_cls_s16),
           cm(t0_cls_s32),
           t0_box_s8, t0_ctr_s8, t1_box_s8, t1_ctr_s8,
           t0_box_s16, t0_ctr_s16, t1_box_s16, t1_ctr_s16,
           t0_box_s32, t0_ctr_s32]
    in_specs = [pl.BlockSpec((_BLK,) + x.shape[1:],
                             (lambda n: (n, 0, 0)) if x.ndim == 3
                             else (lambda n: (n, 0, 0, 0))) for x in ins]
    out_shapes = [jax.ShapeDtypeStruct(x.shape, jnp.float32) for x in ins[5:]]
    out_specs = [pl.BlockSpec((_BLK,) + x.shape[1:], lambda n: (n, 0, 0, 0))
                 for x in ins[5:]]

    prods = pl.pallas_call(
        _body,
        grid=(_B // _BLK,),
        in_specs=in_specs,
        out_specs=out_specs,
        out_shape=out_shapes,
    )(*ins)
    prods = jax.lax.optimization_barrier(tuple(prods))

    result = []
    for g in (prods[0:4], prods[4:8], prods[8:10]):
        for n in range(_B):
            for o in g:
                result.append(o[n])
    return tuple(result)
